# baseline (speedup 1.0000x reference)
import jax
import jax.numpy as jnp
from jax import lax
from jax.experimental import pallas as pl
from jax.experimental.pallas import tpu as pltpu

B, D, C = 16384, 1024, 51
BLK = 4096
NB = B // BLK
NBUF = 2


def _in_copy(x_hbm, xbuf, insem, i):
    return pltpu.make_async_copy(
        x_hbm.at[pl.ds(i * BLK, BLK), :], xbuf.at[i % NBUF], insem.at[i % NBUF])


def _out_copy(pbuf, probs_hbm, outsem, i):
    return pltpu.make_async_copy(
        pbuf.at[i % NBUF], probs_hbm.at[pl.ds(i * BLK, BLK), :],
        outsem.at[i % NBUF])


def _head_kernel(x_hbm, w_ref, b_ref, bins_ref, probs_hbm, val_ref,
                 xbuf, pbuf, insem, outsem):
    for j in range(NBUF):
        _in_copy(x_hbm, xbuf, insem, j).start()
    for i in range(NB):
        j = i % NBUF
        _in_copy(x_hbm, xbuf, insem, i).wait()
        if i >= NBUF:
            _out_copy(pbuf, probs_hbm, outsem, i - NBUF).wait()
        lt = lax.dot_general(
            w_ref[...], xbuf[j],
            (((1,), (1,)), ((), ())),
            preferred_element_type=jnp.float32,
        )
        lt = lt + b_ref[...]
        m = jnp.max(lt, axis=0, keepdims=True)
        e = jnp.exp(lt - m)
        s = jnp.sum(e, axis=0, keepdims=True)
        rinv = 1.0 / s
        num = jnp.sum(e * bins_ref[...], axis=0, keepdims=True)
        pbuf[j] = (e * rinv).T
        val_ref[0, pl.ds(i * BLK, BLK)] = (num * rinv)[0, :]
        _out_copy(pbuf, probs_hbm, outsem, i).start()
        if i + NBUF < NB:
            _in_copy(x_hbm, xbuf, insem, i + NBUF).start()
    for i in range(NB - NBUF, NB):
        _out_copy(pbuf, probs_hbm, outsem, i).wait()


@jax.jit
def kernel(x, W, b, bins):
    b2 = b.reshape(C, 1)
    bins2 = bins.reshape(C, 1)
    probs, val = pl.pallas_call(
        _head_kernel,
        in_specs=[
            pl.BlockSpec(memory_space=pltpu.HBM),
            pl.BlockSpec(memory_space=pltpu.VMEM),
            pl.BlockSpec(memory_space=pltpu.VMEM),
            pl.BlockSpec(memory_space=pltpu.VMEM),
        ],
        out_specs=[
            pl.BlockSpec(memory_space=pltpu.HBM),
            pl.BlockSpec(memory_space=pltpu.VMEM),
        ],
        out_shape=[
            jax.ShapeDtypeStruct((B, C), jnp.float32),
            jax.ShapeDtypeStruct((1, B), jnp.float32),
        ],
        scratch_shapes=[
            pltpu.VMEM((NBUF, BLK, D), jnp.float32),
            pltpu.VMEM((NBUF, BLK, C), jnp.float32),
            pltpu.SemaphoreType.DMA((NBUF,)),
            pltpu.SemaphoreType.DMA((NBUF,)),
        ],
    )(x, W, b2, bins2)
    return probs, val.reshape(B)


# speedup vs baseline: 1.2333x; 1.2333x over previous
import jax
import jax.numpy as jnp
from jax import lax
from jax.experimental import pallas as pl

B, D, C = 16384, 1024, 51


def _head_kernel(x_ref, w_ref, b_ref, bins_ref, pt_ref, val_ref):
    lt = lax.dot_general(
        w_ref[...], x_ref[...],
        (((1,), (1,)), ((), ())),
        preferred_element_type=jnp.float32,
    )
    lt = lt + b_ref[...]
    m = jnp.max(lt, axis=0, keepdims=True)
    e = jnp.exp(lt - m)
    s = jnp.sum(e, axis=0, keepdims=True)
    rinv = 1.0 / s
    num = jnp.sum(e * bins_ref[...], axis=0, keepdims=True)
    pt_ref[...] = e * rinv
    val_ref[0, 0, :] = (num * rinv)[0, :]


@jax.jit
def kernel(x, W, b, bins):
    blk = 1024
    nb = B // blk
    b2 = b.reshape(C, 1)
    bins2 = bins.reshape(C, 1)
    pt, val = pl.pallas_call(
        _head_kernel,
        grid=(nb,),
        in_specs=[
            pl.BlockSpec((blk, D), lambda i: (i, 0)),
            pl.BlockSpec((C, D), lambda i: (0, 0)),
            pl.BlockSpec((C, 1), lambda i: (0, 0)),
            pl.BlockSpec((C, 1), lambda i: (0, 0)),
        ],
        out_specs=[
            pl.BlockSpec((C, blk), lambda i: (0, i)),
            pl.BlockSpec((1, 1, blk), lambda i: (i, 0, 0)),
        ],
        out_shape=[
            jax.ShapeDtypeStruct((C, B), jnp.float32),
            jax.ShapeDtypeStruct((nb, 1, blk), jnp.float32),
        ],
    )(x, W, b2, bins2)
    return pt.T, val.reshape(B)


# speedup vs baseline: 1.4010x; 1.1360x over previous
import jax
import jax.numpy as jnp
from jax import lax
from jax.experimental import pallas as pl

B, D, C = 16384, 1024, 51


def _head(x_ref, w_ref, b_ref, bins_ref, pt_ref, val_ref, col0, blk):
    lt = lax.dot_general(
        w_ref[...], x_ref[...],
        (((1,), (1,)), ((), ())),
        preferred_element_type=jnp.float32,
    )
    lt = lt + b_ref[...]
    m = jnp.max(lt, axis=0, keepdims=True)
    e = jnp.exp(lt - m)
    s = jnp.sum(e, axis=0, keepdims=True)
    rinv = 1.0 / s
    num = jnp.sum(e * bins_ref[...], axis=0, keepdims=True)
    pt_ref[:, col0:col0 + blk] = e * rinv
    val_ref[0, 0, col0:col0 + blk] = (num * rinv)[0, :]


def _head_kernel(x0_ref, x1_ref, w_ref, b_ref, bins_ref, pt_ref, val_ref):
    blk = x0_ref.shape[0]
    _head(x0_ref, w_ref, b_ref, bins_ref, pt_ref, val_ref, 0, blk)
    _head(x1_ref, w_ref, b_ref, bins_ref, pt_ref, val_ref, blk, blk)


@jax.jit
def kernel(x, W, b, bins):
    blk = 1024
    nb = B // blk
    b2 = b.reshape(C, 1)
    bins2 = bins.reshape(C, 1)
    pt, val = pl.pallas_call(
        _head_kernel,
        grid=(nb // 2,),
        in_specs=[
            pl.BlockSpec((blk, D), lambda i: (2 * i, 0)),
            pl.BlockSpec((blk, D), lambda i: (2 * i + 1, 0)),
            pl.BlockSpec((C, D), lambda i: (0, 0)),
            pl.BlockSpec((C, 1), lambda i: (0, 0)),
            pl.BlockSpec((C, 1), lambda i: (0, 0)),
        ],
        out_specs=[
            pl.BlockSpec((C, 2 * blk), lambda i: (0, i)),
            pl.BlockSpec((1, 1, 2 * blk), lambda i: (i, 0, 0)),
        ],
        out_shape=[
            jax.ShapeDtypeStruct((C, B), jnp.float32),
            jax.ShapeDtypeStruct((nb // 2, 1, 2 * blk), jnp.float32),
        ],
    )(x, x, W, b2, bins2)
    return pt.T, val.reshape(B)
